# baseline (speedup 1.0000x reference)
import functools

import jax
import jax.numpy as jnp
import numpy as np
from jax import lax
from jax.experimental import pallas as pl
from jax.experimental.pallas import tpu as pltpu
from jax.experimental.pallas import tpu_sc as plsc

L = 16
NS = 16
ROWS = 16384
D = 128
RPT = ROWS // NS
NG = RPT // L
C = 64
NF = D // L
PB = D + L

_NOISE_VALUES = [
    -0.02830461598932743, 0.4671318531036377, 0.2957029640674591, 0.15354591608047485, -0.12403281778097153, 0.21692314743995667, -1.440878987312317, 0.755859911441803,
    0.5214096307754517, 0.9101703763008118, -0.3844965994358063, 1.139823317527771, 1.4457862377166748, 1.080906629562378, -0.05629321187734604, 0.9095944762229919,
    0.5573461651802063, 0.21905718743801117, -1.4485087394714355, 0.7641875147819519, -0.24154697358608246, -1.179381012916565, -1.9389183521270752, 0.3562646210193634,
    -0.24111966788768768, 1.2151274681091309, -1.3952220678329468, -0.5347688794136047, 0.27067556977272034, 1.5401241779327393, 0.6935186386108398, -0.1038767620921135,
    -0.5023069977760315, 0.6771835088729858, 0.11085006594657898, -0.3477494716644287, 0.45490285754203796, 0.22783830761909485, -0.5570452213287354, -0.8830111026763916,
    -0.21350063383579254, 0.3080112934112549, -0.18721903860569, 0.09363541752099991, 0.3738812208175659, -1.057212471961975, 0.4466709792613983, 1.2107949256896973,
    0.4338840842247009, -0.7037684321403503, 0.17639288306236267, -0.19870367646217346, -0.2181064784526825, 1.2852516174316406, 0.37535151839256287, -0.1780770868062973,
    -0.2397909313440323, -0.4098151624202728, 0.3671177625656128, 1.187896490097046, -1.0384923219680786, -0.7943069338798523, 1.0585581064224243, -0.3621484637260437,
    -0.05511794984340668, -2.0525856018066406, 1.5010137557983398, -1.4625111818313599, 0.08064538985490799, -0.8255164623260498, -0.11807100474834442, -0.9023693203926086,
    0.5638400316238403, -1.0445383787155151, -1.336021065711975, 1.636836051940918, 0.04248049855232239, -1.2391914129257202, -0.18667350709438324, 0.6115323305130005,
    -0.25485995411872864, 1.3313956260681152, 1.0532535314559937, 0.9928337931632996, -1.9690951108932495, -0.52692711353302, -2.3192801475524902, 1.0955307483673096,
    2.4050188064575195, 0.7343149185180664, 0.7297008633613586, -0.9023715257644653, -0.5521381497383118, 0.44048336148262024, -0.4395684003829956, 1.2365392446517944,
    -0.17463453114032745, 0.1723022758960724, 0.2823503911495209, -1.0010589361190796, 0.07685965299606323, 0.8091251254081726, -0.21199345588684082, -2.026014566421509,
    0.562369704246521, 0.8705297112464905, -0.027903152629733086, -1.4850175380706787, -0.7000557780265808, -1.0508149862289429, 0.43867552280426025, 0.7020403146743774,
    -0.39191940426826477, 1.0694249868392944, 0.1372528374195099, -0.45054659247398376, 0.23253656923770905, 0.3512003421783447, 0.5993359088897705, -0.37133026123046875,
    -0.33033689856529236, -0.19157762825489044, -0.14643393456935883, 0.48404356837272644, 1.3645155429840088, -2.144951581954956, 0.4405607581138611, 0.6276503205299377,
]
_NOISE = np.array(_NOISE_VALUES, dtype=np.float32)


def _sc_body(ks_hbm, xs_hbm, k_hbm, noise_hbm, out_hbm,
             ks_v, kv_v, idx_v, rows_v, row0_v, pub_v, noise_v, out_v,
             all_v, shared_v, sem, sem2):
    sid = lax.axis_index("s")
    base = pl.multiple_of(sid * RPT, RPT)

    cp_ks = pltpu.async_copy(ks_hbm.at[pl.ds(base, RPT)], ks_v, sem)
    cp_r0 = pltpu.async_copy(xs_hbm.at[pl.ds(0, 1)], row0_v, sem2)
    pltpu.sync_copy(k_hbm, kv_v)
    kvec = kv_v[...]
    iota = lax.iota(jnp.int32, L)
    cp_ks.wait()

    zi = jnp.zeros((L,), jnp.int32)
    for i in range((RPT + C) // L):
        idx_v[pl.ds(i * L, L)] = zi

    def scan_body(g, off_vec):
        goff = pl.multiple_of(g * L, L)
        vals = ks_v[pl.ds(goff, L)]
        m = vals == kvec
        cnt = plsc.all_reduce_population_count(m)
        pos = plsc.cumsum(m.astype(jnp.int32)) + (off_vec - 1)
        rid = iota + (base + goff)
        plsc.store_scatter(idx_v, [pos], rid, mask=m)
        return off_vec + cnt

    off_vec = lax.fori_loop(0, NG, scan_body, jnp.zeros((L,), jnp.int32))
    n = lax.reduce_max(off_vec, (0,))

    nc = (n + (C - 1)) // C
    zf = jnp.zeros((L,), jnp.float32)

    def chunk_body(c, accs):
        off = pl.multiple_of(c * C, C)
        pltpu.async_copy(xs_hbm.at[idx_v.at[pl.ds(off, C)]], rows_v, sem).wait()
        for j in range(C):
            accs = tuple(accs[f] + rows_v[j, pl.ds(f * L, L)] for f in range(NF))
        return accs

    accs = lax.fori_loop(0, nc, chunk_body, (zf,) * NF)

    cp_r0.wait()
    padv = jnp.full((L,), (nc * C - n).astype(jnp.float32))
    for f in range(NF):
        sl = pl.ds(f * L, L)
        pub_v[sl] = accs[f] - padv * row0_v[0, sl]

    pub_v[pl.ds(D, L)] = off_vec.astype(jnp.float32)
    pltpu.sync_copy(pub_v, shared_v.at[pl.ds(pl.multiple_of(sid * PB, PB), PB)])
    plsc.subcore_barrier()

    @pl.when(sid < NF)
    def _combine():
        cp_all = pltpu.async_copy(shared_v, all_v, sem)
        fl = pl.multiple_of(sid * L, L)
        pltpu.async_copy(noise_hbm.at[pl.ds(fl, L)], noise_v, sem2).wait()
        cp_all.wait()
        tot_cnt = all_v[pl.ds(D, L)]
        s = all_v[pl.ds(fl, L)]
        for t in range(1, NS):
            tot_cnt = tot_cnt + all_v[pl.ds(t * PB + D, L)]
            s = s + all_v[pl.ds(t * PB + fl, L)]
        inv = 1.0 / (tot_cnt + 1.0)
        out_v[...] = (s + noise_v[...]) * inv
        pltpu.sync_copy(out_v, out_hbm.at[pl.ds(fl, L)])


@functools.cache
def _sc_call():
    return pl.kernel(
        _sc_body,
        out_type=jax.ShapeDtypeStruct((D,), jnp.float32),
        mesh=plsc.VectorSubcoreMesh(
            core_axis_name="c", subcore_axis_name="s", num_cores=1, num_subcores=NS),
        compiler_params=pltpu.CompilerParams(needs_layout_passes=False),
        scratch_types=[
            pltpu.VMEM((RPT,), jnp.int32),
            pltpu.VMEM((L,), jnp.int32),
            pltpu.VMEM((RPT + C,), jnp.int32),
            pltpu.VMEM((C, D), jnp.float32),
            pltpu.VMEM((1, D), jnp.float32),
            pltpu.VMEM((PB,), jnp.float32),
            pltpu.VMEM((L,), jnp.float32),
            pltpu.VMEM((L,), jnp.float32),
            pltpu.VMEM((NS * PB,), jnp.float32),
            pltpu.VMEM_SHARED((NS * PB,), jnp.float32),
            pltpu.SemaphoreType.DMA,
            pltpu.SemaphoreType.DMA,
        ],
    )


def kernel(xs, sampled_ks, k):
    k16 = jnp.full((L,), k, dtype=jnp.int32)
    noise = jnp.asarray(_NOISE)
    return _sc_call()(sampled_ks, xs, k16, noise)


# speedup vs baseline: 1.0179x; 1.0179x over previous
import functools

import jax
import jax.numpy as jnp
import numpy as np
from jax import lax
from jax.experimental import pallas as pl
from jax.experimental.pallas import tpu as pltpu
from jax.experimental.pallas import tpu_sc as plsc

L = 16
NS = 16
ROWS = 16384
D = 128
RPT = ROWS // NS
NG = RPT // L
C = 64
NF = D // L
PB = D + L

_NOISE_VALUES = [
    -0.02830461598932743, 0.4671318531036377, 0.2957029640674591, 0.15354591608047485, -0.12403281778097153, 0.21692314743995667, -1.440878987312317, 0.755859911441803,
    0.5214096307754517, 0.9101703763008118, -0.3844965994358063, 1.139823317527771, 1.4457862377166748, 1.080906629562378, -0.05629321187734604, 0.9095944762229919,
    0.5573461651802063, 0.21905718743801117, -1.4485087394714355, 0.7641875147819519, -0.24154697358608246, -1.179381012916565, -1.9389183521270752, 0.3562646210193634,
    -0.24111966788768768, 1.2151274681091309, -1.3952220678329468, -0.5347688794136047, 0.27067556977272034, 1.5401241779327393, 0.6935186386108398, -0.1038767620921135,
    -0.5023069977760315, 0.6771835088729858, 0.11085006594657898, -0.3477494716644287, 0.45490285754203796, 0.22783830761909485, -0.5570452213287354, -0.8830111026763916,
    -0.21350063383579254, 0.3080112934112549, -0.18721903860569, 0.09363541752099991, 0.3738812208175659, -1.057212471961975, 0.4466709792613983, 1.2107949256896973,
    0.4338840842247009, -0.7037684321403503, 0.17639288306236267, -0.19870367646217346, -0.2181064784526825, 1.2852516174316406, 0.37535151839256287, -0.1780770868062973,
    -0.2397909313440323, -0.4098151624202728, 0.3671177625656128, 1.187896490097046, -1.0384923219680786, -0.7943069338798523, 1.0585581064224243, -0.3621484637260437,
    -0.05511794984340668, -2.0525856018066406, 1.5010137557983398, -1.4625111818313599, 0.08064538985490799, -0.8255164623260498, -0.11807100474834442, -0.9023693203926086,
    0.5638400316238403, -1.0445383787155151, -1.336021065711975, 1.636836051940918, 0.04248049855232239, -1.2391914129257202, -0.18667350709438324, 0.6115323305130005,
    -0.25485995411872864, 1.3313956260681152, 1.0532535314559937, 0.9928337931632996, -1.9690951108932495, -0.52692711353302, -2.3192801475524902, 1.0955307483673096,
    2.4050188064575195, 0.7343149185180664, 0.7297008633613586, -0.9023715257644653, -0.5521381497383118, 0.44048336148262024, -0.4395684003829956, 1.2365392446517944,
    -0.17463453114032745, 0.1723022758960724, 0.2823503911495209, -1.0010589361190796, 0.07685965299606323, 0.8091251254081726, -0.21199345588684082, -2.026014566421509,
    0.562369704246521, 0.8705297112464905, -0.027903152629733086, -1.4850175380706787, -0.7000557780265808, -1.0508149862289429, 0.43867552280426025, 0.7020403146743774,
    -0.39191940426826477, 1.0694249868392944, 0.1372528374195099, -0.45054659247398376, 0.23253656923770905, 0.3512003421783447, 0.5993359088897705, -0.37133026123046875,
    -0.33033689856529236, -0.19157762825489044, -0.14643393456935883, 0.48404356837272644, 1.3645155429840088, -2.144951581954956, 0.4405607581138611, 0.6276503205299377,
]
_NOISE = np.array(_NOISE_VALUES, dtype=np.float32)


def _sc_body(ks_hbm, xs_hbm, k_hbm, noise_hbm, out_hbm,
             ks_v, kv_v, idx_v, rows_v, row0_v, pub_v, noise_v, out_v,
             all_v, shared_v, sem, sem2):
    sid = lax.axis_index("s")
    base = pl.multiple_of(sid * RPT, RPT)

    cp_ks = pltpu.async_copy(ks_hbm.at[pl.ds(base, RPT)], ks_v, sem)
    cp_r0 = pltpu.async_copy(xs_hbm.at[pl.ds(0, 1)], row0_v, sem2)
    pltpu.sync_copy(k_hbm, kv_v)
    kvec = kv_v[...]
    iota = lax.iota(jnp.int32, L)
    cp_ks.wait()

    zi = jnp.zeros((L,), jnp.int32)
    for i in range((RPT + C) // L):
        idx_v[pl.ds(i * L, L)] = zi

    def scan_body(g, off_vec):
        goff = pl.multiple_of(g * L, L)
        vals = ks_v[pl.ds(goff, L)]
        m = vals == kvec
        cnt = plsc.all_reduce_population_count(m)
        pos = plsc.cumsum(m.astype(jnp.int32)) + (off_vec - 1)
        rid = iota + (base + goff)
        plsc.store_scatter(idx_v, [pos], rid, mask=m)
        return off_vec + cnt

    with jax.named_scope("p_scan"):
        off_vec = lax.fori_loop(0, NG, scan_body, jnp.zeros((L,), jnp.int32))
        n = lax.reduce_max(off_vec, (0,))

    nc = (n + (C - 1)) // C
    zf = jnp.zeros((L,), jnp.float32)
    for f in range(NF):
        pub_v[pl.ds(f * L, L)] = zf

    def chunk_body(c, _):
        off = pl.multiple_of(c * C, C)
        pltpu.async_copy(xs_hbm.at[idx_v.at[pl.ds(off, C)]], rows_v, sem).wait()
        for j in range(C):
            vals = [rows_v[j, pl.ds(f * L, L)] for f in range(NF)]
            for f in range(NF):
                plsc.addupdate(pub_v.at[pl.ds(f * L, L)], vals[f])
        return 0

    with jax.named_scope("p_gather"):
        lax.fori_loop(0, nc, chunk_body, 0)

    cp_r0.wait()
    padv = jnp.full((L,), (nc * C - n).astype(jnp.float32))
    for f in range(NF):
        sl = pl.ds(f * L, L)
        pub_v[sl] = pub_v[sl] - padv * row0_v[0, sl]

    with jax.named_scope("p_pub"):
        pub_v[pl.ds(D, L)] = off_vec.astype(jnp.float32)
        pltpu.sync_copy(pub_v, shared_v.at[pl.ds(pl.multiple_of(sid * PB, PB), PB)])
        plsc.subcore_barrier()

    @pl.when(sid < NF)
    def _combine():
        cp_all = pltpu.async_copy(shared_v, all_v, sem)
        fl = pl.multiple_of(sid * L, L)
        pltpu.async_copy(noise_hbm.at[pl.ds(fl, L)], noise_v, sem2).wait()
        cp_all.wait()
        tot_cnt = all_v[pl.ds(D, L)]
        s = all_v[pl.ds(fl, L)]
        for t in range(1, NS):
            tot_cnt = tot_cnt + all_v[pl.ds(t * PB + D, L)]
            s = s + all_v[pl.ds(t * PB + fl, L)]
        inv = 1.0 / (tot_cnt + 1.0)
        out_v[...] = (s + noise_v[...]) * inv
        pltpu.sync_copy(out_v, out_hbm.at[pl.ds(fl, L)])


@functools.cache
def _sc_call():
    return pl.kernel(
        _sc_body,
        out_type=jax.ShapeDtypeStruct((D,), jnp.float32),
        mesh=plsc.VectorSubcoreMesh(
            core_axis_name="c", subcore_axis_name="s", num_cores=1, num_subcores=NS),
        compiler_params=pltpu.CompilerParams(needs_layout_passes=False),
        scratch_types=[
            pltpu.VMEM((RPT,), jnp.int32),
            pltpu.VMEM((L,), jnp.int32),
            pltpu.VMEM((RPT + C,), jnp.int32),
            pltpu.VMEM((C, D), jnp.float32),
            pltpu.VMEM((1, D), jnp.float32),
            pltpu.VMEM((PB,), jnp.float32),
            pltpu.VMEM((L,), jnp.float32),
            pltpu.VMEM((L,), jnp.float32),
            pltpu.VMEM((NS * PB,), jnp.float32),
            pltpu.VMEM_SHARED((NS * PB,), jnp.float32),
            pltpu.SemaphoreType.DMA,
            pltpu.SemaphoreType.DMA,
        ],
    )


def kernel(xs, sampled_ks, k):
    k16 = jnp.full((L,), k, dtype=jnp.int32)
    noise = jnp.asarray(_NOISE)
    return _sc_call()(sampled_ks, xs, k16, noise)


# speedup vs baseline: 1.8701x; 1.8372x over previous
import functools

import jax
import jax.numpy as jnp
import numpy as np
from jax import lax
from jax.experimental import pallas as pl
from jax.experimental.pallas import tpu as pltpu
from jax.experimental.pallas import tpu_sc as plsc

L = 16
NS = 16
ROWS = 16384
D = 128
RPT = ROWS // NS
NG = RPT // L
C = 32
NCB = 4
NBUF = C * NCB
NF = D // L
PB = D + L

_NOISE_VALUES = [
    -0.02830461598932743, 0.4671318531036377, 0.2957029640674591, 0.15354591608047485, -0.12403281778097153, 0.21692314743995667, -1.440878987312317, 0.755859911441803,
    0.5214096307754517, 0.9101703763008118, -0.3844965994358063, 1.139823317527771, 1.4457862377166748, 1.080906629562378, -0.05629321187734604, 0.9095944762229919,
    0.5573461651802063, 0.21905718743801117, -1.4485087394714355, 0.7641875147819519, -0.24154697358608246, -1.179381012916565, -1.9389183521270752, 0.3562646210193634,
    -0.24111966788768768, 1.2151274681091309, -1.3952220678329468, -0.5347688794136047, 0.27067556977272034, 1.5401241779327393, 0.6935186386108398, -0.1038767620921135,
    -0.5023069977760315, 0.6771835088729858, 0.11085006594657898, -0.3477494716644287, 0.45490285754203796, 0.22783830761909485, -0.5570452213287354, -0.8830111026763916,
    -0.21350063383579254, 0.3080112934112549, -0.18721903860569, 0.09363541752099991, 0.3738812208175659, -1.057212471961975, 0.4466709792613983, 1.2107949256896973,
    0.4338840842247009, -0.7037684321403503, 0.17639288306236267, -0.19870367646217346, -0.2181064784526825, 1.2852516174316406, 0.37535151839256287, -0.1780770868062973,
    -0.2397909313440323, -0.4098151624202728, 0.3671177625656128, 1.187896490097046, -1.0384923219680786, -0.7943069338798523, 1.0585581064224243, -0.3621484637260437,
    -0.05511794984340668, -2.0525856018066406, 1.5010137557983398, -1.4625111818313599, 0.08064538985490799, -0.8255164623260498, -0.11807100474834442, -0.9023693203926086,
    0.5638400316238403, -1.0445383787155151, -1.336021065711975, 1.636836051940918, 0.04248049855232239, -1.2391914129257202, -0.18667350709438324, 0.6115323305130005,
    -0.25485995411872864, 1.3313956260681152, 1.0532535314559937, 0.9928337931632996, -1.9690951108932495, -0.52692711353302, -2.3192801475524902, 1.0955307483673096,
    2.4050188064575195, 0.7343149185180664, 0.7297008633613586, -0.9023715257644653, -0.5521381497383118, 0.44048336148262024, -0.4395684003829956, 1.2365392446517944,
    -0.17463453114032745, 0.1723022758960724, 0.2823503911495209, -1.0010589361190796, 0.07685965299606323, 0.8091251254081726, -0.21199345588684082, -2.026014566421509,
    0.562369704246521, 0.8705297112464905, -0.027903152629733086, -1.4850175380706787, -0.7000557780265808, -1.0508149862289429, 0.43867552280426025, 0.7020403146743774,
    -0.39191940426826477, 1.0694249868392944, 0.1372528374195099, -0.45054659247398376, 0.23253656923770905, 0.3512003421783447, 0.5993359088897705, -0.37133026123046875,
    -0.33033689856529236, -0.19157762825489044, -0.14643393456935883, 0.48404356837272644, 1.3645155429840088, -2.144951581954956, 0.4405607581138611, 0.6276503205299377,
]
_NOISE = np.array(_NOISE_VALUES, dtype=np.float32)


def _sc_body(ks_hbm, xs_hbm, k_hbm, noise_hbm, out_hbm,
             ks_v, kv_v, idx_v, rows_v, pub_v, noise_v, out_v,
             all_v, shared_v, sem, sem2):
    sid = lax.axis_index("s")
    base = pl.multiple_of(sid * RPT, RPT)

    cp_ks = pltpu.async_copy(ks_hbm.at[pl.ds(base, RPT)], ks_v, sem)
    pltpu.sync_copy(k_hbm, kv_v)
    kvec = kv_v[...]
    iota = lax.iota(jnp.int32, L)
    cp_ks.wait()

    for i in range((RPT + NBUF) // L):
        idx_v[pl.ds(i * L, L)] = iota + (base + (i * L) % RPT)

    def scan_body(g, off_vec):
        goff = pl.multiple_of(g * L, L)
        vals = ks_v[pl.ds(goff, L)]
        m = vals == kvec
        cnt = plsc.all_reduce_population_count(m)
        pos = plsc.cumsum(m.astype(jnp.int32)) + (off_vec - 1)
        rid = iota + (base + goff)
        plsc.store_scatter(idx_v, [pos], rid, mask=m)
        return off_vec + cnt

    with jax.named_scope("p_scan"):
        off_vec = lax.fori_loop(0, NG, scan_body, jnp.zeros((L,), jnp.int32))
        n = lax.reduce_max(off_vec, (0,))

    nc = (n + (C - 1)) // C
    ngs = (nc + (NCB - 1)) // NCB
    zf = jnp.zeros((L,), jnp.float32)
    for f in range(NF):
        pub_v[pl.ds(f * L, L)] = zf

    def sup_body(s, _):
        cbase = s * NCB
        for b in range(NCB):
            @pl.when(cbase + b < nc)
            def _fire():
                off = pl.multiple_of((cbase + b) * C, C)
                pltpu.async_copy(xs_hbm.at[idx_v.at[pl.ds(off, C)]],
                                 rows_v.at[pl.ds(b * C, C), :], sem)
        for b in range(NCB):
            @pl.when(cbase + b < nc)
            def _drain():
                pltpu.make_async_copy(
                    xs_hbm.at[pl.ds(0, C)], rows_v.at[pl.ds(b * C, C), :],
                    sem).wait()
        rem_vec = jnp.full((L,), n - s * NBUF)

        def rbody(j, accs):
            mv = jnp.full((L,), j) < rem_vec
            return tuple(
                accs[f] + jnp.where(mv, rows_v[j, pl.ds(f * L, L)], 0.0)
                for f in range(NF))

        accs = plsc.parallel_loop(0, NBUF, unroll=4, carry=(zf,) * NF)(rbody)
        for f in range(NF):
            plsc.addupdate(pub_v.at[pl.ds(f * L, L)], accs[f])
        return 0

    with jax.named_scope("p_gather"):
        lax.fori_loop(0, ngs, sup_body, 0)

    with jax.named_scope("p_pub"):
        pub_v[pl.ds(D, L)] = off_vec.astype(jnp.float32)
        pltpu.sync_copy(pub_v, shared_v.at[pl.ds(pl.multiple_of(sid * PB, PB), PB)])
        plsc.subcore_barrier()

    @pl.when(sid < NF)
    def _combine():
        cp_all = pltpu.async_copy(shared_v, all_v, sem)
        fl = pl.multiple_of(sid * L, L)
        pltpu.async_copy(noise_hbm.at[pl.ds(fl, L)], noise_v, sem2).wait()
        cp_all.wait()
        tot_cnt = all_v[pl.ds(D, L)]
        s = all_v[pl.ds(fl, L)]
        for t in range(1, NS):
            tot_cnt = tot_cnt + all_v[pl.ds(t * PB + D, L)]
            s = s + all_v[pl.ds(t * PB + fl, L)]
        inv = 1.0 / (tot_cnt + 1.0)
        out_v[...] = (s + noise_v[...]) * inv
        pltpu.sync_copy(out_v, out_hbm.at[pl.ds(fl, L)])


@functools.cache
def _sc_call():
    return pl.kernel(
        _sc_body,
        out_type=jax.ShapeDtypeStruct((D,), jnp.float32),
        mesh=plsc.VectorSubcoreMesh(
            core_axis_name="c", subcore_axis_name="s", num_cores=1, num_subcores=NS),
        compiler_params=pltpu.CompilerParams(needs_layout_passes=False),
        scratch_types=[
            pltpu.VMEM((RPT,), jnp.int32),
            pltpu.VMEM((L,), jnp.int32),
            pltpu.VMEM((RPT + NBUF,), jnp.int32),
            pltpu.VMEM((NBUF, D), jnp.float32),
            pltpu.VMEM((PB,), jnp.float32),
            pltpu.VMEM((L,), jnp.float32),
            pltpu.VMEM((L,), jnp.float32),
            pltpu.VMEM((NS * PB,), jnp.float32),
            pltpu.VMEM_SHARED((NS * PB,), jnp.float32),
            pltpu.SemaphoreType.DMA,
            pltpu.SemaphoreType.DMA,
        ],
    )


def kernel(xs, sampled_ks, k):
    k16 = jnp.full((L,), k, dtype=jnp.int32)
    noise = jnp.asarray(_NOISE)
    return _sc_call()(sampled_ks, xs, k16, noise)


# speedup vs baseline: 1.9233x; 1.0284x over previous
import functools

import jax
import jax.numpy as jnp
import numpy as np
from jax import lax
from jax.experimental import pallas as pl
from jax.experimental.pallas import tpu as pltpu
from jax.experimental.pallas import tpu_sc as plsc

L = 16
NS = 16
ROWS = 16384
D = 128
RPT = ROWS // NS
NG = RPT // L
C = 32
NCB = 4
NBUF = C * NCB
NF = D // L
PB = D + L

_NOISE_VALUES = [
    -0.02830461598932743, 0.4671318531036377, 0.2957029640674591, 0.15354591608047485, -0.12403281778097153, 0.21692314743995667, -1.440878987312317, 0.755859911441803,
    0.5214096307754517, 0.9101703763008118, -0.3844965994358063, 1.139823317527771, 1.4457862377166748, 1.080906629562378, -0.05629321187734604, 0.9095944762229919,
    0.5573461651802063, 0.21905718743801117, -1.4485087394714355, 0.7641875147819519, -0.24154697358608246, -1.179381012916565, -1.9389183521270752, 0.3562646210193634,
    -0.24111966788768768, 1.2151274681091309, -1.3952220678329468, -0.5347688794136047, 0.27067556977272034, 1.5401241779327393, 0.6935186386108398, -0.1038767620921135,
    -0.5023069977760315, 0.6771835088729858, 0.11085006594657898, -0.3477494716644287, 0.45490285754203796, 0.22783830761909485, -0.5570452213287354, -0.8830111026763916,
    -0.21350063383579254, 0.3080112934112549, -0.18721903860569, 0.09363541752099991, 0.3738812208175659, -1.057212471961975, 0.4466709792613983, 1.2107949256896973,
    0.4338840842247009, -0.7037684321403503, 0.17639288306236267, -0.19870367646217346, -0.2181064784526825, 1.2852516174316406, 0.37535151839256287, -0.1780770868062973,
    -0.2397909313440323, -0.4098151624202728, 0.3671177625656128, 1.187896490097046, -1.0384923219680786, -0.7943069338798523, 1.0585581064224243, -0.3621484637260437,
    -0.05511794984340668, -2.0525856018066406, 1.5010137557983398, -1.4625111818313599, 0.08064538985490799, -0.8255164623260498, -0.11807100474834442, -0.9023693203926086,
    0.5638400316238403, -1.0445383787155151, -1.336021065711975, 1.636836051940918, 0.04248049855232239, -1.2391914129257202, -0.18667350709438324, 0.6115323305130005,
    -0.25485995411872864, 1.3313956260681152, 1.0532535314559937, 0.9928337931632996, -1.9690951108932495, -0.52692711353302, -2.3192801475524902, 1.0955307483673096,
    2.4050188064575195, 0.7343149185180664, 0.7297008633613586, -0.9023715257644653, -0.5521381497383118, 0.44048336148262024, -0.4395684003829956, 1.2365392446517944,
    -0.17463453114032745, 0.1723022758960724, 0.2823503911495209, -1.0010589361190796, 0.07685965299606323, 0.8091251254081726, -0.21199345588684082, -2.026014566421509,
    0.562369704246521, 0.8705297112464905, -0.027903152629733086, -1.4850175380706787, -0.7000557780265808, -1.0508149862289429, 0.43867552280426025, 0.7020403146743774,
    -0.39191940426826477, 1.0694249868392944, 0.1372528374195099, -0.45054659247398376, 0.23253656923770905, 0.3512003421783447, 0.5993359088897705, -0.37133026123046875,
    -0.33033689856529236, -0.19157762825489044, -0.14643393456935883, 0.48404356837272644, 1.3645155429840088, -2.144951581954956, 0.4405607581138611, 0.6276503205299377,
]
_NOISE = np.array(_NOISE_VALUES, dtype=np.float32)


def _sc_body(ks_hbm, xs_hbm, k_hbm, noise_hbm, out_hbm,
             ks_v, kv_v, idx_v, rows_v, pub_v, noise_v, out_v,
             all_v, shared_v, sem, sem2):
    sid = lax.axis_index("s")
    base = pl.multiple_of(sid * RPT, RPT)

    cp_ks = pltpu.async_copy(ks_hbm.at[pl.ds(base, RPT)], ks_v, sem)
    cp_kv = pltpu.async_copy(k_hbm, kv_v, sem2)
    iota = lax.iota(jnp.int32, L)

    for i in range((RPT + NBUF) // L):
        idx_v[pl.ds(i * L, L)] = iota + (base + (i * L) % RPT)

    cp_ks.wait()
    cp_kv.wait()
    kvec = kv_v[...]

    def scan_body(g, off_vec):
        goff = pl.multiple_of(g * L, L)
        vals = ks_v[pl.ds(goff, L)]
        m = vals == kvec
        cnt = plsc.all_reduce_population_count(m)
        pos = plsc.cumsum(m.astype(jnp.int32)) + (off_vec - 1)
        rid = iota + (base + goff)
        plsc.store_scatter(idx_v, [pos], rid, mask=m)
        return off_vec + cnt

    with jax.named_scope("p_scan"):
        off_vec = plsc.parallel_loop(
            0, NG, unroll=4, carry=jnp.zeros((L,), jnp.int32))(scan_body)
        n = lax.reduce_max(off_vec, (0,))

    nc = (n + (C - 1)) // C
    ngs = (nc + (NCB - 1)) // NCB
    zf = jnp.zeros((L,), jnp.float32)
    for f in range(NF):
        pub_v[pl.ds(f * L, L)] = zf

    def sup_body(s, _):
        cbase = s * NCB
        for b in range(NCB):
            @pl.when(cbase + b < nc)
            def _fire():
                off = pl.multiple_of((cbase + b) * C, C)
                pltpu.async_copy(xs_hbm.at[idx_v.at[pl.ds(off, C)]],
                                 rows_v.at[pl.ds(b * C, C), :], sem)
        for b in range(NCB):
            @pl.when(cbase + b < nc)
            def _drain():
                pltpu.make_async_copy(
                    xs_hbm.at[pl.ds(0, C)], rows_v.at[pl.ds(b * C, C), :],
                    sem).wait()
        rem_vec = jnp.full((L,), n - s * NBUF)

        def rbody(j, accs):
            mv = jnp.full((L,), j) < rem_vec
            return tuple(
                accs[f] + jnp.where(mv, rows_v[j, pl.ds(f * L, L)], 0.0)
                for f in range(NF))

        accs = plsc.parallel_loop(0, NBUF, unroll=4, carry=(zf,) * NF)(rbody)
        for f in range(NF):
            plsc.addupdate(pub_v.at[pl.ds(f * L, L)], accs[f])
        return 0

    with jax.named_scope("p_gather"):
        lax.fori_loop(0, ngs, sup_body, 0)

    with jax.named_scope("p_pub"):
        pub_v[pl.ds(D, L)] = off_vec.astype(jnp.float32)
        pltpu.sync_copy(pub_v, shared_v.at[pl.ds(pl.multiple_of(sid * PB, PB), PB)])
        plsc.subcore_barrier()

    @pl.when(sid < NF)
    def _combine():
        cp_all = pltpu.async_copy(shared_v, all_v, sem)
        fl = pl.multiple_of(sid * L, L)
        pltpu.async_copy(noise_hbm.at[pl.ds(fl, L)], noise_v, sem2).wait()
        cp_all.wait()
        tot_cnt = all_v[pl.ds(D, L)]
        s = all_v[pl.ds(fl, L)]
        for t in range(1, NS):
            tot_cnt = tot_cnt + all_v[pl.ds(t * PB + D, L)]
            s = s + all_v[pl.ds(t * PB + fl, L)]
        inv = 1.0 / (tot_cnt + 1.0)
        out_v[...] = (s + noise_v[...]) * inv
        pltpu.sync_copy(out_v, out_hbm.at[pl.ds(fl, L)])


@functools.cache
def _sc_call():
    return pl.kernel(
        _sc_body,
        out_type=jax.ShapeDtypeStruct((D,), jnp.float32),
        mesh=plsc.VectorSubcoreMesh(
            core_axis_name="c", subcore_axis_name="s", num_cores=1, num_subcores=NS),
        compiler_params=pltpu.CompilerParams(needs_layout_passes=False),
        scratch_types=[
            pltpu.VMEM((RPT,), jnp.int32),
            pltpu.VMEM((L,), jnp.int32),
            pltpu.VMEM((RPT + NBUF,), jnp.int32),
            pltpu.VMEM((NBUF, D), jnp.float32),
            pltpu.VMEM((PB,), jnp.float32),
            pltpu.VMEM((L,), jnp.float32),
            pltpu.VMEM((L,), jnp.float32),
            pltpu.VMEM((NS * PB,), jnp.float32),
            pltpu.VMEM_SHARED((NS * PB,), jnp.float32),
            pltpu.SemaphoreType.DMA,
            pltpu.SemaphoreType.DMA,
        ],
    )


def kernel(xs, sampled_ks, k):
    k16 = jnp.full((L,), k, dtype=jnp.int32)
    noise = jnp.asarray(_NOISE)
    return _sc_call()(sampled_ks, xs, k16, noise)


# speedup vs baseline: 1.9322x; 1.0046x over previous
import functools

import jax
import jax.numpy as jnp
import numpy as np
from jax import lax
from jax.experimental import pallas as pl
from jax.experimental.pallas import tpu as pltpu
from jax.experimental.pallas import tpu_sc as plsc

L = 16
NS = 16
ROWS = 16384
D = 128
RPT = ROWS // NS
NG = RPT // L
C = 32
NCB = 4
NBUF = C * NCB
NF = D // L
PB = D + L

_NOISE_VALUES = [
    -0.02830461598932743, 0.4671318531036377, 0.2957029640674591, 0.15354591608047485, -0.12403281778097153, 0.21692314743995667, -1.440878987312317, 0.755859911441803,
    0.5214096307754517, 0.9101703763008118, -0.3844965994358063, 1.139823317527771, 1.4457862377166748, 1.080906629562378, -0.05629321187734604, 0.9095944762229919,
    0.5573461651802063, 0.21905718743801117, -1.4485087394714355, 0.7641875147819519, -0.24154697358608246, -1.179381012916565, -1.9389183521270752, 0.3562646210193634,
    -0.24111966788768768, 1.2151274681091309, -1.3952220678329468, -0.5347688794136047, 0.27067556977272034, 1.5401241779327393, 0.6935186386108398, -0.1038767620921135,
    -0.5023069977760315, 0.6771835088729858, 0.11085006594657898, -0.3477494716644287, 0.45490285754203796, 0.22783830761909485, -0.5570452213287354, -0.8830111026763916,
    -0.21350063383579254, 0.3080112934112549, -0.18721903860569, 0.09363541752099991, 0.3738812208175659, -1.057212471961975, 0.4466709792613983, 1.2107949256896973,
    0.4338840842247009, -0.7037684321403503, 0.17639288306236267, -0.19870367646217346, -0.2181064784526825, 1.2852516174316406, 0.37535151839256287, -0.1780770868062973,
    -0.2397909313440323, -0.4098151624202728, 0.3671177625656128, 1.187896490097046, -1.0384923219680786, -0.7943069338798523, 1.0585581064224243, -0.3621484637260437,
    -0.05511794984340668, -2.0525856018066406, 1.5010137557983398, -1.4625111818313599, 0.08064538985490799, -0.8255164623260498, -0.11807100474834442, -0.9023693203926086,
    0.5638400316238403, -1.0445383787155151, -1.336021065711975, 1.636836051940918, 0.04248049855232239, -1.2391914129257202, -0.18667350709438324, 0.6115323305130005,
    -0.25485995411872864, 1.3313956260681152, 1.0532535314559937, 0.9928337931632996, -1.9690951108932495, -0.52692711353302, -2.3192801475524902, 1.0955307483673096,
    2.4050188064575195, 0.7343149185180664, 0.7297008633613586, -0.9023715257644653, -0.5521381497383118, 0.44048336148262024, -0.4395684003829956, 1.2365392446517944,
    -0.17463453114032745, 0.1723022758960724, 0.2823503911495209, -1.0010589361190796, 0.07685965299606323, 0.8091251254081726, -0.21199345588684082, -2.026014566421509,
    0.562369704246521, 0.8705297112464905, -0.027903152629733086, -1.4850175380706787, -0.7000557780265808, -1.0508149862289429, 0.43867552280426025, 0.7020403146743774,
    -0.39191940426826477, 1.0694249868392944, 0.1372528374195099, -0.45054659247398376, 0.23253656923770905, 0.3512003421783447, 0.5993359088897705, -0.37133026123046875,
    -0.33033689856529236, -0.19157762825489044, -0.14643393456935883, 0.48404356837272644, 1.3645155429840088, -2.144951581954956, 0.4405607581138611, 0.6276503205299377,
]
_NOISE = np.array(_NOISE_VALUES, dtype=np.float32)


def _sc_body(ks_hbm, xs_hbm, k_hbm, noise_hbm, out_hbm,
             ks_v, kv_v, idx_v, rows_v, pub_v, noise_v, out_v,
             all_v, shared_v, sem, sem2):
    sid = lax.axis_index("s")
    base = pl.multiple_of(sid * RPT, RPT)

    cp_ks = pltpu.async_copy(ks_hbm.at[pl.ds(base, RPT)], ks_v, sem)
    cp_kv = pltpu.async_copy(k_hbm, kv_v, sem2)
    iota = lax.iota(jnp.int32, L)

    for i in range((RPT + NBUF) // L):
        idx_v[pl.ds(i * L, L)] = iota + (base + (i * L) % RPT)

    cp_ks.wait()
    cp_kv.wait()
    kvec = kv_v[...]

    @pl.when(sid < NF)
    def _noise_prefetch():
        fl = pl.multiple_of(sid * L, L)
        pltpu.async_copy(noise_hbm.at[pl.ds(fl, L)], noise_v, sem2)

    def scan_body(g, off_vec):
        goff = pl.multiple_of(g * L, L)
        vals = ks_v[pl.ds(goff, L)]
        m = vals == kvec
        cnt = plsc.all_reduce_population_count(m)
        pos = plsc.cumsum(m.astype(jnp.int32)) + (off_vec - 1)
        rid = iota + (base + goff)
        plsc.store_scatter(idx_v, [pos], rid, mask=m)
        return off_vec + cnt

    with jax.named_scope("p_scan"):
        off_vec = plsc.parallel_loop(
            0, NG, unroll=4, carry=jnp.zeros((L,), jnp.int32))(scan_body)
        n = lax.reduce_max(off_vec, (0,))

    nc = (n + (C - 1)) // C
    ngs = (nc + (NCB - 1)) // NCB
    zf = jnp.zeros((L,), jnp.float32)
    for f in range(NF):
        pub_v[pl.ds(f * L, L)] = zf

    def sup_body(s, _):
        cbase = s * NCB
        for b in range(NCB):
            @pl.when(cbase + b < nc)
            def _fire():
                off = pl.multiple_of((cbase + b) * C, C)
                pltpu.async_copy(xs_hbm.at[idx_v.at[pl.ds(off, C)]],
                                 rows_v.at[pl.ds(b * C, C), :], sem)
        for b in range(NCB):
            @pl.when(cbase + b < nc)
            def _drain():
                pltpu.make_async_copy(
                    xs_hbm.at[pl.ds(0, C)], rows_v.at[pl.ds(b * C, C), :],
                    sem).wait()
        rem_vec = jnp.full((L,), n - s * NBUF)

        def rbody(j, accs):
            mv = jnp.full((L,), j) < rem_vec
            return tuple(
                accs[f] + jnp.where(mv, rows_v[j, pl.ds(f * L, L)], 0.0)
                for f in range(NF))

        accs = plsc.parallel_loop(0, NBUF, unroll=8, carry=(zf,) * NF)(rbody)
        for f in range(NF):
            plsc.addupdate(pub_v.at[pl.ds(f * L, L)], accs[f])
        return 0

    with jax.named_scope("p_gather"):
        lax.fori_loop(0, ngs, sup_body, 0)

    with jax.named_scope("p_pub"):
        pub_v[pl.ds(D, L)] = off_vec.astype(jnp.float32)
        pltpu.sync_copy(pub_v, shared_v.at[pl.ds(pl.multiple_of(sid * PB, PB), PB)])
        plsc.subcore_barrier()

    @pl.when(sid < NF)
    def _combine():
        cp_all = pltpu.async_copy(shared_v, all_v, sem)
        fl = pl.multiple_of(sid * L, L)
        pltpu.make_async_copy(
            noise_hbm.at[pl.ds(fl, L)], noise_v, sem2).wait()
        cp_all.wait()
        tot_cnt = all_v[pl.ds(D, L)]
        s = all_v[pl.ds(fl, L)]
        for t in range(1, NS):
            tot_cnt = tot_cnt + all_v[pl.ds(t * PB + D, L)]
            s = s + all_v[pl.ds(t * PB + fl, L)]
        inv = 1.0 / (tot_cnt + 1.0)
        out_v[...] = (s + noise_v[...]) * inv
        pltpu.sync_copy(out_v, out_hbm.at[pl.ds(fl, L)])


@functools.cache
def _sc_call():
    return pl.kernel(
        _sc_body,
        out_type=jax.ShapeDtypeStruct((D,), jnp.float32),
        mesh=plsc.VectorSubcoreMesh(
            core_axis_name="c", subcore_axis_name="s", num_cores=1, num_subcores=NS),
        compiler_params=pltpu.CompilerParams(needs_layout_passes=False),
        scratch_types=[
            pltpu.VMEM((RPT,), jnp.int32),
            pltpu.VMEM((L,), jnp.int32),
            pltpu.VMEM((RPT + NBUF,), jnp.int32),
            pltpu.VMEM((NBUF, D), jnp.float32),
            pltpu.VMEM((PB,), jnp.float32),
            pltpu.VMEM((L,), jnp.float32),
            pltpu.VMEM((L,), jnp.float32),
            pltpu.VMEM((NS * PB,), jnp.float32),
            pltpu.VMEM_SHARED((NS * PB,), jnp.float32),
            pltpu.SemaphoreType.DMA,
            pltpu.SemaphoreType.DMA,
        ],
    )


def kernel(xs, sampled_ks, k):
    k16 = jnp.full((L,), k, dtype=jnp.int32)
    noise = jnp.asarray(_NOISE)
    return _sc_call()(sampled_ks, xs, k16, noise)
